# Initial kernel scaffold; baseline (speedup 1.0000x reference)
import jax
import jax.numpy as jnp
from jax.experimental import pallas as pl


def kernel(x_G, x_R, src_GR, dst_GR, ei_RR, Wk, bk, Wq, bq, Wv, bv, Ws, bs, Wg, bg, gamma, beta, layer_weights, M):
    raise NotImplementedError("write your pallas kernel here")


# speedup vs baseline: 21.5348x; 21.5348x over previous
import functools
import jax
import jax.numpy as jnp
from jax import lax
from jax.experimental import pallas as pl
from jax.experimental.pallas import tpu as pltpu
from jax.experimental.pallas import tpu_sc as plsc

B = 2
NG = 2000
NR = 10000
GE = 16
RE = 128
L = 2
E_GR = 80000
E_RR = 160000

E_GR_PAD = 81920
SUBS = 16
ROWS_PER_SUB = NR // SUBS

_MESH = plsc.VectorSubcoreMesh(core_axis_name="c", subcore_axis_name="s")


def _stripe(sub):
    start = sub * 640
    nchunks = jnp.where(sub < 15, 8, 5)
    return start, nchunks


def _zero_stripe(acc, zbuf, sub, width):
    zrows = zbuf.shape[0]

    def zloop(i, _):
        for j in range(width // 16):
            zbuf[i, pl.ds(j * 16, 16)] = jnp.zeros((16,), jnp.float32)
        return 0

    lax.fori_loop(0, zrows, zloop, 0)
    start, nchunks = _stripe(sub)

    def cp(i, _):
        pltpu.sync_copy(zbuf, acc.at[pl.ds(start + i * 80, 80)])
        return 0

    lax.fori_loop(0, nchunks, cp, 0)


def _write_stripe(acc, out_hbm, sub, row_off):
    start, nchunks = _stripe(sub)

    def cp(i, _):
        pltpu.sync_copy(acc.at[pl.ds(start + i * 80, 80)],
                        out_hbm.at[pl.ds(row_off + start + i * 80, 80)])
        return 0

    lax.fori_loop(0, nchunks, cp, 0)


def _deg_body(dst_hbm, out_hbm, dst_v, ones_v, zbuf, hist, sem):
    cidx = lax.axis_index("c")
    sub = lax.axis_index("s")
    C = 80
    nchunks = E_RR // (SUBS * C)

    _zero_stripe(hist, zbuf, sub, RE)

    def oloop(i, _):
        for j in range(RE // 16):
            ones_v[i, pl.ds(j * 16, 16)] = jnp.ones((16,), jnp.float32)
        return 0

    lax.fori_loop(0, C, oloop, 0)
    plsc.subcore_barrier()

    def chunk(i, _):
        base = sub * (E_RR // SUBS) + i * C
        pltpu.sync_copy(dst_hbm.at[pl.ds(base, C)], dst_v)
        pltpu.sync_copy(ones_v, hist.at[dst_v], add=True)
        return 0

    lax.fori_loop(0, nchunks, chunk, 0)
    plsc.subcore_barrier()
    _write_stripe(hist, out_hbm, sub, cidx * NR)


_deg_kernel = pl.kernel(
    _deg_body,
    out_type=jax.ShapeDtypeStruct((B * NR, RE), jnp.float32),
    mesh=_MESH,
    scratch_types=[
        pltpu.VMEM((80,), jnp.int32),
        pltpu.VMEM((80, RE), jnp.float32),
        pltpu.VMEM((80, RE), jnp.float32),
        pltpu.VMEM_SHARED((NR, RE), jnp.float32),
        pltpu.SemaphoreType.DMA,
    ],
)


def _rr_body(y_hbm, src_hbm, dst_hbm, out_hbm,
             src_v, dst_v, ybuf, zbuf, acc, sem):
    cidx = lax.axis_index("c")
    sub = lax.axis_index("s")
    C = 80
    nchunks = E_RR // (SUBS * C)

    _zero_stripe(acc, zbuf, sub, RE)
    plsc.subcore_barrier()

    def chunk(i, _):
        base = sub * (E_RR // SUBS) + i * C
        pltpu.sync_copy(src_hbm.at[pl.ds(cidx * E_RR + base, C)], src_v)
        pltpu.sync_copy(dst_hbm.at[pl.ds(base, C)], dst_v)
        pltpu.async_copy(y_hbm.at[src_v], ybuf, sem).wait()
        pltpu.sync_copy(ybuf, acc.at[dst_v], add=True)
        return 0

    lax.fori_loop(0, nchunks, chunk, 0)
    plsc.subcore_barrier()
    _write_stripe(acc, out_hbm, sub, cidx * NR)


_rr_kernel = pl.kernel(
    _rr_body,
    out_type=jax.ShapeDtypeStruct((B * NR, RE), jnp.float32),
    mesh=_MESH,
    scratch_types=[
        pltpu.VMEM((80,), jnp.int32),
        pltpu.VMEM((80,), jnp.int32),
        pltpu.VMEM((80, RE), jnp.float32),
        pltpu.VMEM((80, RE), jnp.float32),
        pltpu.VMEM_SHARED((NR, RE), jnp.float32),
        pltpu.SemaphoreType.DMA,
    ],
)


def _gr_body(k_hbm, q_hbm, v_hbm, kidx_hbm, qidx_hbm, dst_hbm, out_hbm,
             kidx_v, qidx_v, dst_v, kbuf, qbuf, vbuf, zbuf, acc, sem):
    cidx = lax.axis_index("c")
    sub = lax.axis_index("s")
    C = 80
    nchunks = E_GR_PAD // (SUBS * C)

    _zero_stripe(acc, zbuf, sub, RE)
    plsc.subcore_barrier()

    def chunk(i, _):
        base = sub * (E_GR_PAD // SUBS) + i * C
        pltpu.sync_copy(kidx_hbm.at[pl.ds(cidx * E_GR_PAD + base, C)], kidx_v)
        pltpu.sync_copy(qidx_hbm.at[pl.ds(cidx * E_GR_PAD + base, C)], qidx_v)
        pltpu.sync_copy(dst_hbm.at[pl.ds(base, C)], dst_v)
        cp_k = pltpu.async_copy(k_hbm.at[kidx_v], kbuf, sem)
        cp_q = pltpu.async_copy(q_hbm.at[qidx_v], qbuf, sem)
        cp_v = pltpu.async_copy(v_hbm.at[qidx_v], vbuf, sem)
        cp_k.wait()
        cp_q.wait()
        cp_v.wait()

        def edge(e, _):
            for j in range(RE // 16):
                sl = pl.ds(j * 16, 16)
                x = kbuf[e, sl] + qbuf[e, sl]
                kbuf[e, sl] = vbuf[e, sl] / (1.0 + jnp.exp(-x))
            return 0

        lax.fori_loop(0, C, edge, 0)
        pltpu.sync_copy(kbuf, acc.at[dst_v], add=True)
        return 0

    lax.fori_loop(0, nchunks, chunk, 0)
    plsc.subcore_barrier()
    _write_stripe(acc, out_hbm, sub, cidx * NR)


_gr_kernel = pl.kernel(
    _gr_body,
    out_type=jax.ShapeDtypeStruct((B * NR, RE), jnp.float32),
    mesh=_MESH,
    scratch_types=[
        pltpu.VMEM((80,), jnp.int32),
        pltpu.VMEM((80,), jnp.int32),
        pltpu.VMEM((80,), jnp.int32),
        pltpu.VMEM((80, RE), jnp.float32),
        pltpu.VMEM((80, RE), jnp.float32),
        pltpu.VMEM((80, RE), jnp.float32),
        pltpu.VMEM((80, RE), jnp.float32),
        pltpu.VMEM_SHARED((NR + 8, RE), jnp.float32),
        pltpu.SemaphoreType.DMA,
    ],
)


def _qv_body(xg_ref, wq_ref, bq_ref, wv_ref, bv_ref, q_ref, v_ref):
    x = xg_ref[...]
    for l in range(L):
        q_ref[l] = jnp.dot(x, wq_ref[l], preferred_element_type=jnp.float32) + bq_ref[l]
        v_ref[l] = jnp.dot(x, wv_ref[l], preferred_element_type=jnp.float32) + bv_ref[l]


def _qv_call(xg2, Wq, bq, Wv, bv):
    return pl.pallas_call(
        _qv_body,
        out_shape=[jax.ShapeDtypeStruct((L, B * NG, RE), jnp.float32),
                   jax.ShapeDtypeStruct((L, B * NG, RE), jnp.float32)],
    )(xg2, Wq, bq, Wv, bv)


_RB = 1000
_NBLK = (B * NR) // _RB


def _prep0_body(lw_ref, x_ref, hist_ref, wk_ref, bk_ref, wg_ref, ws_ref, bsg_ref,
                k_ref, y_ref, s_ref, dis_ref, rr_ref):
    x = x_ref[...]
    deg = hist_ref[0, :, 0:1] + 1.0
    dis = lax.rsqrt(jnp.maximum(deg, 1e-12))
    dis_ref[...] = dis
    k_ref[...] = jnp.dot(x, wk_ref[...], preferred_element_type=jnp.float32) + bk_ref[...]
    y = jnp.dot(x, wg_ref[...], preferred_element_type=jnp.float32) * dis
    y_ref[...] = y
    s_ref[...] = (jnp.dot(x, ws_ref[...], preferred_element_type=jnp.float32)
                  + bsg_ref[...] + y * dis)
    rr_ref[...] = lw_ref[0] * x


def _prep0_call(lwts, x2, hist2, Wk0, bk0, Wg0, Ws0, bsg0):
    row_spec = pl.BlockSpec((_RB, RE), lambda i: (i, 0))
    full128 = pl.BlockSpec((RE, RE), lambda i: (0, 0))
    bias = pl.BlockSpec((1, RE), lambda i: (0, 0))
    return pl.pallas_call(
        _prep0_body,
        grid=(_NBLK,),
        in_specs=[
            pl.BlockSpec(memory_space=pltpu.SMEM),
            row_spec,
            pl.BlockSpec((B, _RB, RE), lambda i: (0, i % (NR // _RB), 0)),
            full128, bias, full128, full128, bias,
        ],
        out_specs=[
            row_spec, row_spec, row_spec,
            pl.BlockSpec((_RB, 1), lambda i: (i, 0)),
            row_spec,
        ],
        out_shape=[
            jax.ShapeDtypeStruct((B * NR, RE), jnp.float32),
            jax.ShapeDtypeStruct((B * NR, RE), jnp.float32),
            jax.ShapeDtypeStruct((B * NR, RE), jnp.float32),
            jax.ShapeDtypeStruct((B * NR, 1), jnp.float32),
            jax.ShapeDtypeStruct((B * NR, RE), jnp.float32),
        ],
    )(lwts, x2, hist2, Wk0, bk0, Wg0, Ws0, bsg0)


def _gelu_ln(pre, gamma, beta):
    g = 0.5 * pre * (1.0 + lax.erf(pre * 0.7071067811865476))
    mu = jnp.mean(g, axis=-1, keepdims=True)
    d = g - mu
    var = jnp.mean(d * d, axis=-1, keepdims=True)
    return d * lax.rsqrt(var + 1e-5) * gamma + beta


def _comb0_body(lw_ref, agg_ref, acc_ref, skip_ref, dis_ref, gam_ref, bet_ref,
                rrp_ref, wk_ref, bk_ref, wg_ref, ws_ref, bsg_ref,
                rr_ref, k_ref, y_ref, s_ref):
    dis = dis_ref[...]
    pre = agg_ref[...] + acc_ref[...] * dis + skip_ref[...]
    x1 = _gelu_ln(pre, gam_ref[...], bet_ref[...])
    rr_ref[...] = rrp_ref[...] + lw_ref[1] * x1
    k_ref[...] = jnp.dot(x1, wk_ref[...], preferred_element_type=jnp.float32) + bk_ref[...]
    y = jnp.dot(x1, wg_ref[...], preferred_element_type=jnp.float32) * dis
    y_ref[...] = y
    s_ref[...] = (jnp.dot(x1, ws_ref[...], preferred_element_type=jnp.float32)
                  + bsg_ref[...] + y * dis)


def _comb0_call(lwts, agg, acc, skip, dis, gam, bet, rrp, Wk1, bk1, Wg1, Ws1, bsg1):
    row_spec = pl.BlockSpec((_RB, RE), lambda i: (i, 0))
    full128 = pl.BlockSpec((RE, RE), lambda i: (0, 0))
    bias = pl.BlockSpec((1, RE), lambda i: (0, 0))
    dis_spec = pl.BlockSpec((_RB, 1), lambda i: (i, 0))
    return pl.pallas_call(
        _comb0_body,
        grid=(_NBLK,),
        in_specs=[
            pl.BlockSpec(memory_space=pltpu.SMEM),
            row_spec, row_spec, row_spec, dis_spec, bias, bias, row_spec,
            full128, bias, full128, full128, bias,
        ],
        out_specs=[row_spec, row_spec, row_spec, row_spec],
        out_shape=[
            jax.ShapeDtypeStruct((B * NR, RE), jnp.float32),
            jax.ShapeDtypeStruct((B * NR, RE), jnp.float32),
            jax.ShapeDtypeStruct((B * NR, RE), jnp.float32),
            jax.ShapeDtypeStruct((B * NR, RE), jnp.float32),
        ],
    )(lwts, agg, acc, skip, dis, gam, bet, rrp, Wk1, bk1, Wg1, Ws1, bsg1)


def _final_body(lw_ref, agg_ref, acc_ref, skip_ref, dis_ref, gam_ref, bet_ref,
                rrp_ref, m_ref, flx_ref):
    dis = dis_ref[...]
    pre = agg_ref[...] + acc_ref[...] * dis + skip_ref[...]
    x2 = _gelu_ln(pre, gam_ref[...], bet_ref[...])
    rr = rrp_ref[...] + lw_ref[2] * x2
    t1 = rr[:, : RE // 2]
    t2 = rr[:, RE // 2:]
    m = m_ref[...]
    a = jnp.sum(t1 * lax.dot_general(t2, m, (((1,), (1,)), ((), ())),
                                     preferred_element_type=jnp.float32),
                axis=-1, keepdims=True)
    b = jnp.sum(t2 * lax.dot_general(t1, m, (((1,), (1,)), ((), ())),
                                     preferred_element_type=jnp.float32),
                axis=-1, keepdims=True)
    flx_ref[...] = a - b


def _final_call(lwts, agg, acc, skip, dis, gam, bet, rrp, M):
    row_spec = pl.BlockSpec((_RB, RE), lambda i: (i, 0))
    bias = pl.BlockSpec((1, RE), lambda i: (0, 0))
    dis_spec = pl.BlockSpec((_RB, 1), lambda i: (i, 0))
    return pl.pallas_call(
        _final_body,
        grid=(_NBLK,),
        in_specs=[
            pl.BlockSpec(memory_space=pltpu.SMEM),
            row_spec, row_spec, row_spec, dis_spec, bias, bias, row_spec,
            pl.BlockSpec((RE // 2, RE // 2), lambda i: (0, 0)),
        ],
        out_specs=pl.BlockSpec((_RB, 1), lambda i: (i, 0)),
        out_shape=jax.ShapeDtypeStruct((B * NR, 1), jnp.float32),
    )(lwts, agg, acc, skip, dis, gam, bet, rrp, M)


def kernel(x_G, x_R, src_GR, dst_GR, ei_RR, Wk, bk, Wq, bq, Wv, bv, Ws, bs,
           Wg, bg, gamma, beta, layer_weights, M):
    lwts = jax.nn.softmax(layer_weights)
    src_RR = ei_RR[0]
    dst_RR = ei_RR[1]

    npad = E_GR_PAD - E_GR
    src_gr = jnp.concatenate([src_GR, jnp.zeros((npad,), jnp.int32)])
    dst_gr = jnp.concatenate([dst_GR, jnp.full((npad,), NR, jnp.int32)])
    dstk = jnp.minimum(dst_gr, NR - 1)
    kidx_gr = jnp.concatenate([dstk, dstk + NR])
    qidx_gr = jnp.concatenate([src_gr, src_gr + NG])
    srcall_RR = jnp.concatenate([src_RR, src_RR + NR])

    x2 = x_R.reshape(B * NR, RE)
    xg2 = x_G.reshape(B * NG, GE)

    hist = _deg_kernel(dst_RR)
    hist2 = hist.reshape(B, NR, RE)

    q_all, v_all = _qv_call(xg2, Wq, bq.reshape(L, 1, RE), Wv, bv.reshape(L, 1, RE))

    bsg = [(bs[l] + bg[l]).reshape(1, RE) for l in range(L)]
    k0, y0, s0, dis, rr0 = _prep0_call(
        lwts, x2, hist2, Wk[0], bk[0].reshape(1, RE), Wg[0], Ws[0], bsg[0])

    agg0 = _gr_kernel(k0, q_all[0], v_all[0], kidx_gr, qidx_gr, dst_gr)
    acc0 = _rr_kernel(y0, srcall_RR, dst_RR)

    rr1, k1, y1, s1 = _comb0_call(
        lwts, agg0, acc0, s0, dis, gamma[0].reshape(1, RE), beta[0].reshape(1, RE),
        rr0, Wk[1], bk[1].reshape(1, RE), Wg[1], Ws[1], bsg[1])

    agg1 = _gr_kernel(k1, q_all[1], v_all[1], kidx_gr, qidx_gr, dst_gr)
    acc1 = _rr_kernel(y1, srcall_RR, dst_RR)

    flx = _final_call(lwts, agg1, acc1, s1, dis, gamma[1].reshape(1, RE),
                      beta[1].reshape(1, RE), rr1, M)
    return flx.reshape(B, NR)
